# baseline (speedup 1.0000x reference)
import functools

import jax
import jax.numpy as jnp
from jax import lax
from jax.experimental import pallas as pl
from jax.experimental.pallas import tpu as pltpu
from jax.experimental.pallas import tpu_sc as plsc

NC = 2
NS = 16
NW = NC * NS
LANES = 16
BBLK = 128
VPAD = 128


@jax.jit
def kernel(x, token_table, pos_table):
    B, T = x.shape
    V, D = token_table.shape

    xT = x.T
    ttp = jnp.pad(token_table, ((0, 0), (0, VPAD - D)))
    posT = pos_table.T
    PCOL = 256

    mesh = plsc.VectorSubcoreMesh(core_axis_name="c", subcore_axis_name="s")

    @functools.partial(
        pl.kernel,
        mesh=mesh,
        compiler_params=pltpu.CompilerParams(
            use_tc_tiling_on_sc=True, needs_layout_passes=False),
        out_type=jax.ShapeDtypeStruct((T, D, B), jnp.float32),
        scratch_types=[
            pltpu.VMEM((T, BBLK), jnp.int32),
            pltpu.VMEM((D, PCOL), jnp.float32),
            [pltpu.VMEM((BBLK, VPAD), jnp.float32)] * 2,
            [pltpu.VMEM((D, BBLK), jnp.float32)] * 2,
            [pltpu.SemaphoreType.DMA] * 2,
            [pltpu.SemaphoreType.DMA] * 2,
            pltpu.SemaphoreType.DMA,
        ],
    )
    def emb(xT_hbm, tt_hbm, posT_hbm, out_hbm, idx_v, pos_v, gbufs, obufs,
            gsems, osems, sem0):
        wid = lax.axis_index("s") * NC + lax.axis_index("c")
        bcol = pl.multiple_of(wid * BBLK, BBLK)

        cp_pos = pltpu.async_copy(posT_hbm.at[:, pl.ds(0, PCOL)], pos_v, sem0)
        cp_idx = pltpu.async_copy(xT_hbm.at[:, pl.ds(bcol, BBLK)], idx_v, sem0)
        cp_pos.wait()
        cp_idx.wait()

        def issue_gather(t, j):
            pltpu.async_copy(tt_hbm.at[idx_v.at[t]], gbufs[j], gsems[j])

        def wait_gather(t, j):
            pltpu.make_async_copy(
                tt_hbm.at[idx_v.at[t]], gbufs[j], gsems[j]).wait()

        def issue_out(t, j):
            pltpu.async_copy(
                obufs[j], out_hbm.at[t, :, pl.ds(bcol, BBLK)], osems[j])

        def wait_out(t, j):
            pltpu.make_async_copy(
                obufs[j], out_hbm.at[t, :, pl.ds(bcol, BBLK)], osems[j]).wait()

        issue_gather(0, 0)
        issue_gather(1, 1)

        iotas = [lax.iota(jnp.int32, LANES) + (g * LANES)
                 for g in range(BBLK // LANES)]

        @pl.loop(0, T, step=2)
        def _(t0):
            for j in range(2):
                t = t0 + j

                wait_gather(t, j)

                @pl.when(t >= 2)
                def _():
                    wait_out(t - 2, j)

                tvec = jnp.full((LANES,), t, jnp.int32)

                @pl.loop(0, D)
                def _(f):
                    cidx = jnp.full((LANES,), f, jnp.int32)
                    s = plsc.load_gather(pos_v, [cidx, tvec])
                    for g in range(BBLK // LANES):
                        v = plsc.load_gather(gbufs[j], [iotas[g], cidx])
                        obufs[j][f, pl.ds(g * LANES, LANES)] = v + s

                issue_out(t, j)

                @pl.when(t + 2 < T)
                def _():
                    issue_gather(t + 2, j)

        wait_out(T - 2, 0)
        wait_out(T - 1, 1)

    out = emb(xT, ttp, posT)
    return out.transpose(2, 0, 1)
